# baseline (speedup 1.0000x reference)
import jax
import jax.numpy as jnp
from jax import lax
from jax.experimental import pallas as pl
from jax.experimental.pallas import tpu as pltpu

_NGROUPS = 4


def _balanced_groups(l_dim, extra_rows_g0):
    ls = list(range(2, l_dim + 1))
    total = sum(ls) + extra_rows_g0
    groups, cur, acc = [], [], extra_rows_g0
    target = total / _NGROUPS
    for l in ls:
        cur.append(l)
        acc += l
        if acc >= target * (len(groups) + 1) and len(groups) < _NGROUPS - 1:
            groups.append(cur)
            cur = []
    groups.append(cur)
    while len(groups) < _NGROUPS:
        groups.append([])
    return groups


def _make_body(a_dim, l_dim, groups):
    def body(x1_ref, x2_ref, w_ref, b_ref, zero_ref, scores_ref, *rest):
        out_refs = rest[:l_dim - 1]
        stash_ref = rest[l_dim - 1]
        j = pl.program_id(1)

        @pl.when(j == 0)
        def _compute():
            x1 = x1_ref[...]
            x2 = x2_ref[...]
            w1 = w_ref[:a_dim, :]
            w2 = w_ref[a_dim:, :]
            st = lax.dot_general(w1, x1, (((0,), (1,)), ((), ())),
                                 preferred_element_type=jnp.float32)
            st = st + lax.dot_general(w2, x2, (((0,), (0,)), ((), ())),
                                      preferred_element_type=jnp.float32)
            st = st + jnp.transpose(b_ref[...])

            zero_ref[...] = jnp.zeros_like(zero_ref)
            scores_ref[...] = jnp.transpose(st)

            tk = st.shape[1]
            m_row = jnp.max(st, axis=0, keepdims=True)
            cs = jnp.exp(st - m_row)
            shift = 1
            while shift < l_dim:
                shifted = jnp.concatenate(
                    [jnp.zeros((shift, tk), jnp.float32), cs[:-shift, :]],
                    axis=0,
                )
                cs = cs + shifted
                shift *= 2
            clse = m_row + jnp.log(jnp.maximum(cs, jnp.float32(1e-37)))
            stash_ref[:l_dim, :] = st
            stash_ref[l_dim:, :] = clse

        st = stash_ref[:l_dim, :]
        clse = stash_ref[l_dim:, :]
        for g, ls in enumerate(groups):
            if not ls:
                continue

            @pl.when(j == g)
            def _write(ls=ls):
                for l in ls:
                    out_refs[l - 2][...] = st[:l, :] - clse[l - 1:l, :]

    return body


def _pick_tile(k_total):
    for t in (2048, 1024, 512, 256, 128, 64, 32, 16, 8):
        if k_total % t == 0:
            return t
    return k_total


def kernel(A_from, A_to, W, b):
    k_total, a_dim = A_from.shape
    l_dim = W.shape[1]
    tile_k = _pick_tile(k_total)
    grid = (k_total // tile_k, _NGROUPS)

    groups = _balanced_groups(l_dim, extra_rows_g0=l_dim + 1)

    out_shape = [
        jax.ShapeDtypeStruct((1, k_total), jnp.float32),
        jax.ShapeDtypeStruct((k_total, l_dim), jnp.float32),
    ]
    out_specs = [
        pl.BlockSpec((1, tile_k), lambda i, j: (0, i)),
        pl.BlockSpec((tile_k, l_dim), lambda i, j: (i, 0)),
    ]
    for l in range(2, l_dim + 1):
        out_shape.append(jax.ShapeDtypeStruct((l, k_total), jnp.float32))
        out_specs.append(pl.BlockSpec((l, tile_k), lambda i, j: (0, i)))

    zrow, scores, *lps = pl.pallas_call(
        _make_body(a_dim, l_dim, groups),
        grid=grid,
        out_shape=tuple(out_shape),
        in_specs=[
            pl.BlockSpec((tile_k, a_dim), lambda i, j: (i, 0)),
            pl.BlockSpec((a_dim, tile_k), lambda i, j: (0, i)),
            pl.BlockSpec((2 * a_dim, l_dim), lambda i, j: (0, 0)),
            pl.BlockSpec((1, l_dim), lambda i, j: (0, 0)),
        ],
        out_specs=tuple(out_specs),
        scratch_shapes=[
            pltpu.VMEM((2 * l_dim, tile_k), jnp.float32),
        ],
        compiler_params=pltpu.CompilerParams(
            dimension_semantics=("parallel", "arbitrary"),
            vmem_limit_bytes=56 * 1024 * 1024,
        ),
    )(A_from, A_to, W, b.astype(jnp.float32))

    lplist = [zrow] + lps
    return lplist, scores


# speedup vs baseline: 1.2440x; 1.2440x over previous
import jax
import jax.numpy as jnp
from jax import lax
from jax.experimental import pallas as pl
from jax.experimental.pallas import tpu as pltpu


def _fused_kernel(x1_ref, x2_ref, w_ref, b_ref, zero_ref, scores_ref, *out_refs):
    x1 = x1_ref[...]
    x2 = x2_ref[...]
    a_dim = x1.shape[1]
    w1 = w_ref[:a_dim, :]
    w2 = w_ref[a_dim:, :]

    st = lax.dot_general(w1, x1, (((0,), (1,)), ((), ())),
                         preferred_element_type=jnp.float32)
    st = st + lax.dot_general(w2, x2, (((0,), (0,)), ((), ())),
                              preferred_element_type=jnp.float32)
    st = st + jnp.transpose(b_ref[...])

    zero_ref[...] = jnp.zeros_like(zero_ref)
    scores_ref[...] = jnp.transpose(st)

    ll, tk = st.shape
    m_row = jnp.max(st, axis=0, keepdims=True)
    cs = jnp.exp(st - m_row)
    shift = 1
    while shift < ll:
        shifted = jnp.concatenate(
            [jnp.zeros((shift, tk), jnp.float32), cs[:-shift, :]], axis=0
        )
        cs = cs + shifted
        shift *= 2
    clse = m_row + jnp.log(jnp.maximum(cs, jnp.float32(1e-37)))

    for idx, l in enumerate(range(2, ll + 1)):
        out_refs[idx][...] = st[:l, :] - clse[l - 1:l, :]


def _pick_tile(k_total):
    for t in (2048, 1024, 512, 256, 128, 64, 32, 16, 8):
        if k_total % t == 0:
            return t
    return k_total


def kernel(A_from, A_to, W, b):
    k_total, a_dim = A_from.shape
    l_dim = W.shape[1]
    tile_k = _pick_tile(k_total)
    grid = (k_total // tile_k,)

    out_shape = [
        jax.ShapeDtypeStruct((1, k_total), jnp.float32),
        jax.ShapeDtypeStruct((k_total, l_dim), jnp.float32),
    ]
    out_specs = [
        pl.BlockSpec((1, tile_k), lambda i: (0, i)),
        pl.BlockSpec((tile_k, l_dim), lambda i: (i, 0)),
    ]
    for l in range(2, l_dim + 1):
        out_shape.append(jax.ShapeDtypeStruct((l, k_total), jnp.float32))
        out_specs.append(pl.BlockSpec((l, tile_k), lambda i: (0, i)))

    zrow, scores, *lps = pl.pallas_call(
        _fused_kernel,
        grid=grid,
        out_shape=tuple(out_shape),
        in_specs=[
            pl.BlockSpec((tile_k, a_dim), lambda i: (i, 0)),
            pl.BlockSpec((a_dim, tile_k), lambda i: (0, i)),
            pl.BlockSpec((2 * a_dim, l_dim), lambda i: (0, 0)),
            pl.BlockSpec((1, l_dim), lambda i: (0, 0)),
        ],
        out_specs=tuple(out_specs),
        compiler_params=pltpu.CompilerParams(
            dimension_semantics=("parallel",),
            vmem_limit_bytes=56 * 1024 * 1024,
        ),
    )(A_from, A_to, W, b.astype(jnp.float32))

    lplist = [zrow] + lps
    return lplist, scores
